# baseline (speedup 1.0000x reference)
import jax
import jax.numpy as jnp
from jax import lax
from jax.experimental import pallas as pl
from jax.experimental.pallas import tpu as pltpu
from jax.experimental.pallas import tpu_sc as plsc

VOCAB = 1_000_000
EMB = 64
SEQ = 200
BATCH = 4096
SCALE = 8.0

NC = 2
NS = 16
NW = NC * NS
LANES = 16

NSLAB = VOCAB // 128
SLAB_ITERS = 246
TAIL0 = NSLAB * 128
B_PER_W = BATCH // NW

_PARAMS = pltpu.CompilerParams(
    use_tc_tiling_on_sc=True, needs_layout_passes=False)
_MESH = dict(core_axis_name="c", subcore_axis_name="s")


def _repack_body(tabT, tail, packed, ib0, ib1, ob0, ob1, tb,
                 si0, si1, so0, so1):
    wid = lax.axis_index("s") * NC + lax.axis_index("c")
    ns = 244 + jnp.where(wid < 4, 1, 0)
    ibs, obs = (ib0, ib1), (ob0, ob1)
    si, so = (si0, si1), (so0, so1)
    iota = lax.iota(jnp.int32, LANES)
    rowv = [c * LANES + iota for c in range(4)]

    def slab(i):
        return wid + 32 * jnp.where(i < ns, i, 0)

    def issue_in(i, b):
        pltpu.async_copy(tabT.at[:, pl.ds(128 * slab(i), 128)], ibs[b], si[b])

    def drain_in(i, b):
        pltpu.make_async_copy(
            tabT.at[:, pl.ds(128 * slab(i), 128)], ibs[b], si[b]).wait()

    def issue_out(i, b):
        pltpu.async_copy(obs[b], packed.at[pl.ds(64 * slab(i), 64)], so[b])

    def drain_out(i, b):
        pltpu.make_async_copy(
            obs[b], packed.at[pl.ds(64 * slab(i), 64)], so[b]).wait()

    def transpose(b, buf, nrow):
        def j_body(j, carry):
            for c in range(8):
                p = c // 4
                colv = jnp.full((LANES,), 2 * j + p, jnp.int32)
                v = plsc.load_gather(buf, [rowv[c % 4], colv])
                obs[b][j, pl.ds(c * LANES, LANES)] = v
            return carry

        lax.fori_loop(0, nrow, j_body, 0)

    issue_in(0, 0)
    issue_in(1, 1)

    def outer(ii, carry):
        for b in range(2):
            i = ii * 2 + b
            drain_in(i, b)

            @pl.when(i >= 2)
            def _():
                drain_out(i - 2, b)

            transpose(b, ibs[b], 64)
            issue_out(i, b)

            @pl.when(i + 2 < SLAB_ITERS)
            def _():
                issue_in(i + 2, b)
        return carry

    lax.fori_loop(0, SLAB_ITERS // 2, outer, 0)
    drain_out(SLAB_ITERS - 2, 0)
    drain_out(SLAB_ITERS - 1, 1)

    @pl.when(wid == 31)
    def _():
        pltpu.sync_copy(tail, tb)
        transpose(0, tb, 32)
        pltpu.sync_copy(obs[0].at[pl.ds(0, 32)],
                        packed.at[pl.ds(TAIL0 // 2, 32)])


def _repack(token_table):
    fn = pl.kernel(
        _repack_body,
        out_type=jax.ShapeDtypeStruct((VOCAB // 2, 128), jnp.float32),
        mesh=plsc.VectorSubcoreMesh(**_MESH),
        compiler_params=_PARAMS,
        scratch_types=[
            pltpu.VMEM((EMB, 128), jnp.float32),
            pltpu.VMEM((EMB, 128), jnp.float32),
            pltpu.VMEM((EMB, 128), jnp.float32),
            pltpu.VMEM((EMB, 128), jnp.float32),
            pltpu.VMEM((EMB, EMB), jnp.float32),
            pltpu.SemaphoreType.DMA,
            pltpu.SemaphoreType.DMA,
            pltpu.SemaphoreType.DMA,
            pltpu.SemaphoreType.DMA,
        ],
    )
    return fn(jnp.transpose(token_table),
              jnp.transpose(token_table[TAIL0:]))


def _gather_body(packed, src, pos1, out3,
                 idxT, stage, g0, g1, o0, o1, x0, x1, pos_v,
                 sg0, sg1, so0, so1):
    wid = lax.axis_index("s") * NC + lax.axis_index("c")
    b0 = wid * B_PER_W
    gb, ob, xb = (g0, g1), (o0, o1), (x0, x1)
    sg, so = (sg0, sg1), (so0, so1)
    iota = lax.iota(jnp.int32, LANES)
    rowv = [c * LANES + iota for c in range(8)]

    pltpu.sync_copy(pos1, pos_v)

    for c in range(8):
        pltpu.sync_copy(src.at[pl.ds(b0 + c * LANES, LANES)], stage)

        def s_body(s, carry):
            v = plsc.load_gather(stage, [iota, jnp.full((LANES,), s, jnp.int32)])
            idxT[s, pl.ds(c * LANES, LANES)] = v
            return carry

        lax.fori_loop(0, SEQ, s_body, 0)

    def prep_idx2(s, b):
        for k in range(8):
            xb[b][pl.ds(k * LANES, LANES)] = lax.shift_right_logical(
                idxT[s, pl.ds(k * LANES, LANES)], 1)

    def issue_gather(b):
        pltpu.async_copy(packed.at[xb[b]], gb[b], sg[b])

    def drain_gather(b):
        pltpu.make_async_copy(packed.at[xb[b]], gb[b], sg[b]).wait()

    def issue_out(s, b):
        pltpu.async_copy(ob[b], out3.at[s, :, pl.ds(b0, B_PER_W)], so[b])

    def drain_out(s, b):
        pltpu.make_async_copy(
            ob[b], out3.at[s, :, pl.ds(b0, B_PER_W)], so[b]).wait()

    def compute(s, b):
        colbase = []
        for c in range(8):
            pv = lax.bitwise_and(idxT[s, pl.ds(c * LANES, LANES)], 1)
            colbase.append(pv * EMB)

        def e_body(e, carry):
            posv = plsc.load_gather(
                pos_v, [jnp.full((LANES,), s * EMB + e, jnp.int32)])
            for c in range(8):
                v = plsc.load_gather(gb[b], [rowv[c], colbase[c] + e])
                ob[b][e, pl.ds(c * LANES, LANES)] = v * SCALE + posv
            return carry

        lax.fori_loop(0, EMB, e_body, 0)

    prep_idx2(0, 0)
    issue_gather(0)
    prep_idx2(1, 1)
    issue_gather(1)

    def outer(i, carry):
        for b in range(2):
            s = i * 2 + b
            drain_gather(b)

            @pl.when(s >= 2)
            def _():
                drain_out(s - 2, b)

            compute(s, b)
            issue_out(s, b)

            @pl.when(s + 2 < SEQ)
            def _():
                prep_idx2(s + 2, b)
                issue_gather(b)
        return carry

    lax.fori_loop(0, SEQ // 2, outer, 0)
    drain_out(SEQ - 2, 0)
    drain_out(SEQ - 1, 1)


def _gather(packed, src, pos1):
    fn = pl.kernel(
        _gather_body,
        out_type=jax.ShapeDtypeStruct((SEQ, EMB, BATCH), jnp.float32),
        mesh=plsc.VectorSubcoreMesh(**_MESH),
        compiler_params=_PARAMS,
        scratch_types=[
            pltpu.VMEM((SEQ, 128), jnp.int32),
            pltpu.VMEM((LANES, SEQ), jnp.int32),
            pltpu.VMEM((128, 128), jnp.float32),
            pltpu.VMEM((128, 128), jnp.float32),
            pltpu.VMEM((EMB, 128), jnp.float32),
            pltpu.VMEM((EMB, 128), jnp.float32),
            pltpu.VMEM((128,), jnp.int32),
            pltpu.VMEM((128,), jnp.int32),
            pltpu.VMEM((SEQ * EMB,), jnp.float32),
            pltpu.SemaphoreType.DMA,
            pltpu.SemaphoreType.DMA,
            pltpu.SemaphoreType.DMA,
            pltpu.SemaphoreType.DMA,
        ],
    )
    return fn(packed, src, pos1)


@jax.jit
def _emb_lookup(src, token_table, pos_table):
    packed = _repack(token_table)
    out3 = _gather(packed, src, pos_table.reshape(SEQ * EMB))
    return jnp.transpose(out3, (2, 0, 1))


def kernel(src, tgt, token_table, pos_table):
    del tgt
    return _emb_lookup(src, token_table, pos_table)


# speedup vs baseline: 1.3538x; 1.3538x over previous
import jax
import jax.numpy as jnp
from jax import lax
from jax.experimental import pallas as pl
from jax.experimental.pallas import tpu as pltpu
from jax.experimental.pallas import tpu_sc as plsc

VOCAB = 1_000_000
EMB = 64
SEQ = 200
BATCH = 4096
SCALE = 8.0

NC = 2
NS = 16
NW = NC * NS
LANES = 16

NSLAB = VOCAB // 128
SLAB_ITERS = 246
TAIL0 = NSLAB * 128
B_PER_W = BATCH // NW

_PARAMS = pltpu.CompilerParams(
    use_tc_tiling_on_sc=True, needs_layout_passes=False)
_MESH = dict(core_axis_name="c", subcore_axis_name="s")


def _repack_body(tabT, tail, packed, ib0, ib1, ob0, ob1, tb,
                 si0, si1, so0, so1):
    wid = lax.axis_index("s") * NC + lax.axis_index("c")
    ns = 244 + jnp.where(wid < 4, 1, 0)
    ibs, obs = (ib0, ib1), (ob0, ob1)
    si, so = (si0, si1), (so0, so1)
    iota = lax.iota(jnp.int32, LANES)
    rowv = [c * LANES + iota for c in range(4)]

    def slab(i):
        return wid + 32 * jnp.where(i < ns, i, 0)

    def issue_in(i, b):
        pltpu.async_copy(tabT.at[:, pl.ds(128 * slab(i), 128)], ibs[b], si[b])

    def drain_in(i, b):
        pltpu.make_async_copy(
            tabT.at[:, pl.ds(128 * slab(i), 128)], ibs[b], si[b]).wait()

    def issue_out(i, b):
        pltpu.async_copy(obs[b], packed.at[pl.ds(64 * slab(i), 64)], so[b])

    def drain_out(i, b):
        pltpu.make_async_copy(
            obs[b], packed.at[pl.ds(64 * slab(i), 64)], so[b]).wait()

    def transpose(b, buf, nrow):
        def j_body(j, carry):
            col0 = jnp.full((LANES,), 2 * j, jnp.int32)
            col1 = col0 + 1
            vs = []
            for c in range(8):
                colv = col1 if c >= 4 else col0
                vs.append(plsc.load_gather(buf, [rowv[c % 4], colv]))
            for c in range(8):
                obs[b][j, pl.ds(c * LANES, LANES)] = vs[c]
            return carry

        lax.fori_loop(0, nrow, j_body, 0)

    issue_in(0, 0)
    issue_in(1, 1)

    def outer(ii, carry):
        for b in range(2):
            i = ii * 2 + b
            drain_in(i, b)

            @pl.when(i >= 2)
            def _():
                drain_out(i - 2, b)

            transpose(b, ibs[b], 64)
            issue_out(i, b)

            @pl.when(i + 2 < SLAB_ITERS)
            def _():
                issue_in(i + 2, b)
        return carry

    lax.fori_loop(0, SLAB_ITERS // 2, outer, 0)
    drain_out(SLAB_ITERS - 2, 0)
    drain_out(SLAB_ITERS - 1, 1)

    @pl.when(wid == 31)
    def _():
        pltpu.sync_copy(tail, tb)
        transpose(0, tb, 32)
        pltpu.sync_copy(obs[0].at[pl.ds(0, 32)],
                        packed.at[pl.ds(TAIL0 // 2, 32)])


def _repack(token_table):
    fn = pl.kernel(
        _repack_body,
        out_type=jax.ShapeDtypeStruct((VOCAB // 2, 128), jnp.float32),
        mesh=plsc.VectorSubcoreMesh(**_MESH),
        compiler_params=_PARAMS,
        scratch_types=[
            pltpu.VMEM((EMB, 128), jnp.float32),
            pltpu.VMEM((EMB, 128), jnp.float32),
            pltpu.VMEM((EMB, 128), jnp.float32),
            pltpu.VMEM((EMB, 128), jnp.float32),
            pltpu.VMEM((EMB, EMB), jnp.float32),
            pltpu.SemaphoreType.DMA,
            pltpu.SemaphoreType.DMA,
            pltpu.SemaphoreType.DMA,
            pltpu.SemaphoreType.DMA,
        ],
    )
    return fn(jnp.transpose(token_table),
              jnp.transpose(token_table[TAIL0:]))


_BLK = [0, 16, 32, 48, 64, 80, 96, 112, 128, 144, 160, 176, 184]


def _gather_body(packed, src, pos1, out_hbm,
                 ix0, ix1, x20, x21, pb0, pb1, g0, g1, o0, o1, pos_v,
                 si0, si1, sg0, sg1, so0, so1):
    wid = lax.axis_index("s") * NC + lax.axis_index("c")
    row0 = wid * B_PER_W
    ix, x2, pb = (ix0, ix1), (x20, x21), (pb0, pb1)
    gb, ob = (g0, g1), (o0, o1)
    si, sg, so = (si0, si1), (sg0, sg1), (so0, so1)

    pltpu.sync_copy(pos1, pos_v)

    def issue_idx(i, b):
        pltpu.async_copy(src.at[row0 + i], ix[b], si[b])

    def drain_idx(b):
        pltpu.make_async_copy(src.at[row0], ix[b], si[b]).wait()

    def shift(b):
        for k in range(13):
            o = min(k * LANES, SEQ - LANES)
            v = ix[b][pl.ds(o, LANES)]
            x2[b][pl.ds(o, LANES)] = lax.shift_right_logical(v, 1)
            pb[b][pl.ds(o, LANES)] = lax.bitwise_and(v, 1)

    def issue_gather(b):
        pltpu.async_copy(packed.at[x2[b].at[pl.ds(0, 104)]],
                         gb[b].at[pl.ds(0, 104)], sg[b])
        pltpu.async_copy(packed.at[x2[b].at[pl.ds(104, 96)]],
                         gb[b].at[pl.ds(104, 96)], sg[b])

    def drain_gather(b):
        pltpu.make_async_copy(packed.at[x2[b].at[pl.ds(0, 104)]],
                              gb[b].at[pl.ds(0, 104)], sg[b]).wait()
        pltpu.make_async_copy(packed.at[x2[b].at[pl.ds(104, 96)]],
                              gb[b].at[pl.ds(104, 96)], sg[b]).wait()

    def issue_out(i, b):
        pltpu.async_copy(ob[b], out_hbm.at[row0 + i], so[b])

    def drain_out(i, b):
        pltpu.make_async_copy(ob[b], out_hbm.at[row0 + i], so[b]).wait()

    def compute(b):
        def blk_body(k, carry):
            o = pl.multiple_of(jnp.minimum(k * LANES, SEQ - LANES), 8)
            pv = pb[b][pl.ds(o, LANES)]
            for lane in range(LANES):
                r = o + lane
                po = pl.multiple_of(pv[lane] * EMB, LANES)
                for e in range(4):
                    tv = gb[b][r, pl.ds(po + e * LANES, LANES)]
                    pvv = pos_v[pl.ds(
                        pl.multiple_of(r * EMB + e * LANES, LANES), LANES)]
                    ob[b][r, pl.ds(e * LANES, LANES)] = tv * SCALE + pvv
            return carry

        lax.fori_loop(0, 13, blk_body, 0)

    issue_idx(0, 0)
    issue_idx(1, 1)
    drain_idx(0)
    shift(0)
    issue_gather(0)
    drain_idx(1)
    shift(1)
    issue_gather(1)

    def outer(ii, carry):
        for b in range(2):
            i = ii * 2 + b
            drain_gather(b)

            @pl.when(i >= 2)
            def _():
                drain_out(i - 2, b)

            compute(b)
            issue_out(i, b)

            @pl.when(i + 2 < B_PER_W)
            def _():
                issue_idx(i + 2, b)

            bo = 1 - b

            @pl.when(jnp.logical_and(i >= 1, i + 1 < B_PER_W))
            def _():
                drain_idx(bo)
                shift(bo)
                issue_gather(bo)
        return carry

    lax.fori_loop(0, B_PER_W // 2, outer, 0)
    drain_out(B_PER_W - 2, 0)
    drain_out(B_PER_W - 1, 1)


def _gather(packed, src, pos1):
    fn = pl.kernel(
        _gather_body,
        out_type=jax.ShapeDtypeStruct((BATCH, SEQ, EMB), jnp.float32),
        mesh=plsc.VectorSubcoreMesh(**_MESH),
        compiler_params=_PARAMS,
        scratch_types=[
            pltpu.VMEM((SEQ,), jnp.int32),
            pltpu.VMEM((SEQ,), jnp.int32),
            pltpu.VMEM((SEQ,), jnp.int32),
            pltpu.VMEM((SEQ,), jnp.int32),
            pltpu.VMEM((SEQ,), jnp.int32),
            pltpu.VMEM((SEQ,), jnp.int32),
            pltpu.VMEM((SEQ, 128), jnp.float32),
            pltpu.VMEM((SEQ, 128), jnp.float32),
            pltpu.VMEM((SEQ, EMB), jnp.float32),
            pltpu.VMEM((SEQ, EMB), jnp.float32),
            pltpu.VMEM((SEQ * EMB,), jnp.float32),
            pltpu.SemaphoreType.DMA,
            pltpu.SemaphoreType.DMA,
            pltpu.SemaphoreType.DMA,
            pltpu.SemaphoreType.DMA,
            pltpu.SemaphoreType.DMA,
            pltpu.SemaphoreType.DMA,
        ],
    )
    return fn(packed, src, pos1)


@jax.jit
def _emb_lookup(src, token_table, pos_table):
    packed = _repack(token_table)
    return _gather(packed, src, pos_table.reshape(SEQ * EMB))


def kernel(src, tgt, token_table, pos_table):
    del tgt
    return _emb_lookup(src, token_table, pos_table)


# speedup vs baseline: 1.6635x; 1.2288x over previous
import jax
import jax.numpy as jnp
from jax import lax
from jax.experimental import pallas as pl
from jax.experimental.pallas import tpu as pltpu
from jax.experimental.pallas import tpu_sc as plsc

VOCAB = 1_000_000
EMB = 64
SEQ = 200
BATCH = 4096
SCALE = 8.0

NC = 2
NS = 16
NW = NC * NS
LANES = 16

NSLAB = VOCAB // 128
SLAB_ITERS = 246
TAIL0 = NSLAB * 128
B_PER_W = BATCH // NW

_PARAMS = pltpu.CompilerParams(
    use_tc_tiling_on_sc=True, needs_layout_passes=False)
_MESH = dict(core_axis_name="c", subcore_axis_name="s")


def _repack_body(tabT, tail, packed, ib0, ib1, ob0, ob1, tb,
                 si0, si1, so0, so1):
    wid = lax.axis_index("s") * NC + lax.axis_index("c")
    ns = 244 + jnp.where(wid < 4, 1, 0)
    ibs, obs = (ib0, ib1), (ob0, ob1)
    si, so = (si0, si1), (so0, so1)
    iota = lax.iota(jnp.int32, LANES)
    rowv = [c * LANES + iota for c in range(4)]

    def slab(i):
        return wid + 32 * jnp.where(i < ns, i, 0)

    def issue_in(i, b):
        pltpu.async_copy(tabT.at[:, pl.ds(128 * slab(i), 128)], ibs[b], si[b])

    def drain_in(i, b):
        pltpu.make_async_copy(
            tabT.at[:, pl.ds(128 * slab(i), 128)], ibs[b], si[b]).wait()

    def issue_out(i, b):
        pltpu.async_copy(obs[b], packed.at[pl.ds(64 * slab(i), 64)], so[b])

    def drain_out(i, b):
        pltpu.make_async_copy(
            obs[b], packed.at[pl.ds(64 * slab(i), 64)], so[b]).wait()

    def transpose(b, buf, nrow):
        def j_body(j, carry):
            col0 = jnp.full((LANES,), 2 * j, jnp.int32)
            col1 = col0 + 1
            vs = []
            for c in range(8):
                colv = col1 if c >= 4 else col0
                vs.append(plsc.load_gather(buf, [rowv[c % 4], colv]))
            for c in range(8):
                obs[b][j, pl.ds(c * LANES, LANES)] = vs[c]
            return carry

        lax.fori_loop(0, nrow, j_body, 0)

    issue_in(0, 0)
    issue_in(1, 1)

    def outer(ii, carry):
        for b in range(2):
            i = ii * 2 + b
            drain_in(i, b)

            @pl.when(i >= 2)
            def _():
                drain_out(i - 2, b)

            transpose(b, ibs[b], 64)
            issue_out(i, b)

            @pl.when(i + 2 < SLAB_ITERS)
            def _():
                issue_in(i + 2, b)
        return carry

    lax.fori_loop(0, SLAB_ITERS // 2, outer, 0)
    drain_out(SLAB_ITERS - 2, 0)
    drain_out(SLAB_ITERS - 1, 1)

    @pl.when(wid == 31)
    def _():
        pltpu.sync_copy(tail, tb)
        transpose(0, tb, 32)
        pltpu.sync_copy(obs[0].at[pl.ds(0, 32)],
                        packed.at[pl.ds(TAIL0 // 2, 32)])


def _repack(token_table):
    fn = pl.kernel(
        _repack_body,
        out_type=jax.ShapeDtypeStruct((VOCAB // 2, 128), jnp.float32),
        mesh=plsc.VectorSubcoreMesh(**_MESH),
        compiler_params=_PARAMS,
        scratch_types=[
            pltpu.VMEM((EMB, 128), jnp.float32),
            pltpu.VMEM((EMB, 128), jnp.float32),
            pltpu.VMEM((EMB, 128), jnp.float32),
            pltpu.VMEM((EMB, 128), jnp.float32),
            pltpu.VMEM((EMB, EMB), jnp.float32),
            pltpu.SemaphoreType.DMA,
            pltpu.SemaphoreType.DMA,
            pltpu.SemaphoreType.DMA,
            pltpu.SemaphoreType.DMA,
        ],
    )
    return fn(jnp.transpose(token_table),
              jnp.transpose(token_table[TAIL0:]))


_BLK = [0, 16, 32, 48, 64, 80, 96, 112, 128, 144, 160, 176, 184]


def _gather_body(packed, src, pos1, out_hbm,
                 ix0, ix1, x20, x21, pb0, pb1, g0, g1, o0, o1, pos_v,
                 si0, si1, sg0, sg1, so0, so1):
    wid = lax.axis_index("s") * NC + lax.axis_index("c")
    row0 = wid * B_PER_W
    ix, x2, pb = (ix0, ix1), (x20, x21), (pb0, pb1)
    gb, ob = (g0, g1), (o0, o1)
    si, sg, so = (si0, si1), (sg0, sg1), (so0, so1)

    pltpu.sync_copy(pos1, pos_v)

    def issue_idx(i, b):
        pltpu.async_copy(src.at[row0 + i], ix[b], si[b])

    def drain_idx(b):
        pltpu.make_async_copy(src.at[row0], ix[b], si[b]).wait()

    def shift(b):
        for k in range(13):
            o = min(k * LANES, SEQ - LANES)
            v = ix[b][pl.ds(o, LANES)]
            x2[b][pl.ds(o, LANES)] = lax.shift_right_logical(v, 1)
            pb[b][pl.ds(o, LANES)] = lax.bitwise_and(v, 1)

    def issue_gather(b):
        pltpu.async_copy(packed.at[x2[b].at[pl.ds(0, 104)]],
                         gb[b].at[pl.ds(0, 104)], sg[b])
        pltpu.async_copy(packed.at[x2[b].at[pl.ds(104, 96)]],
                         gb[b].at[pl.ds(104, 96)], sg[b])

    def drain_gather(b):
        pltpu.make_async_copy(packed.at[x2[b].at[pl.ds(0, 104)]],
                              gb[b].at[pl.ds(0, 104)], sg[b]).wait()
        pltpu.make_async_copy(packed.at[x2[b].at[pl.ds(104, 96)]],
                              gb[b].at[pl.ds(104, 96)], sg[b]).wait()

    def issue_out(i, b):
        pltpu.async_copy(ob[b], out_hbm.at[row0 + i], so[b])

    def drain_out(i, b):
        pltpu.make_async_copy(ob[b], out_hbm.at[row0 + i], so[b]).wait()

    def compute(b):
        def blk_body(k, carry):
            o = pl.multiple_of(jnp.minimum(k * LANES, SEQ - LANES), 8)
            pv = pb[b][pl.ds(o, LANES)]
            for lane in range(LANES):
                r = o + lane
                po = pl.multiple_of(pv[lane] * EMB, LANES)
                for e in range(4):
                    tv = gb[b][r, pl.ds(po + e * LANES, LANES)]
                    pvv = pos_v[pl.ds(
                        pl.multiple_of(r * EMB + e * LANES, LANES), LANES)]
                    ob[b][r, pl.ds(e * LANES, LANES)] = tv * SCALE + pvv
            return carry

        lax.fori_loop(0, 13, blk_body, 0)

    issue_idx(0, 0)
    issue_idx(1, 1)
    drain_idx(0)
    shift(0)
    issue_gather(0)
    drain_idx(1)
    shift(1)
    issue_gather(1)

    def outer(ii, carry):
        for b in range(2):
            i = ii * 2 + b
            drain_gather(b)

            @pl.when(i >= 2)
            def _():
                drain_out(i - 2, b)

            compute(b)
            issue_out(i, b)

            @pl.when(i + 2 < B_PER_W)
            def _():
                issue_idx(i + 2, b)

            bo = 1 - b

            @pl.when(jnp.logical_and(i >= 1, i + 1 < B_PER_W))
            def _():
                drain_idx(bo)
                shift(bo)
                issue_gather(bo)
        return carry

    lax.fori_loop(0, B_PER_W // 2, outer, 0)
    drain_out(B_PER_W - 2, 0)
    drain_out(B_PER_W - 1, 1)


def _gather(packed, src, pos1):
    fn = pl.kernel(
        _gather_body,
        out_type=jax.ShapeDtypeStruct((BATCH, SEQ, EMB), jnp.float32),
        mesh=plsc.VectorSubcoreMesh(**_MESH),
        compiler_params=_PARAMS,
        scratch_types=[
            pltpu.VMEM((SEQ,), jnp.int32),
            pltpu.VMEM((SEQ,), jnp.int32),
            pltpu.VMEM((SEQ,), jnp.int32),
            pltpu.VMEM((SEQ,), jnp.int32),
            pltpu.VMEM((SEQ,), jnp.int32),
            pltpu.VMEM((SEQ,), jnp.int32),
            pltpu.VMEM((SEQ, 128), jnp.float32),
            pltpu.VMEM((SEQ, 128), jnp.float32),
            pltpu.VMEM((SEQ, EMB), jnp.float32),
            pltpu.VMEM((SEQ, EMB), jnp.float32),
            pltpu.VMEM((SEQ * EMB,), jnp.float32),
            pltpu.SemaphoreType.DMA,
            pltpu.SemaphoreType.DMA,
            pltpu.SemaphoreType.DMA,
            pltpu.SemaphoreType.DMA,
            pltpu.SemaphoreType.DMA,
            pltpu.SemaphoreType.DMA,
        ],
    )
    return fn(packed, src, pos1)


@jax.jit
def _emb_lookup(src, token_table, pos_table):
    packed = token_table.reshape(VOCAB // 2, 128)
    return _gather(packed, src, pos_table.reshape(SEQ * EMB))


def kernel(src, tgt, token_table, pos_table):
    del tgt
    return _emb_lookup(src, token_table, pos_table)


# speedup vs baseline: 2.4291x; 1.4602x over previous
import jax
import jax.numpy as jnp
from jax import lax
from jax.experimental import pallas as pl
from jax.experimental.pallas import tpu as pltpu
from jax.experimental.pallas import tpu_sc as plsc

EMB = 64
SEQ = 200
SCALE = 8.0

NC = 2
NS = 16
NW = NC * NS
LANES = 16
NREG = EMB // LANES

BATCH = 4096
B_PER_W = BATCH // NW
G = 2
NG = B_PER_W // G


def _body(src_hbm, tok_hbm, pos_hbm, out_hbm,
          ix0, ix1, ib0, ib1, ob0, ob1, pos_v,
          si0, si1, sg0, sg1, ss0, ss1):
    wid = lax.axis_index("s") * NC + lax.axis_index("c")
    row0 = wid * B_PER_W

    pltpu.sync_copy(pos_hbm, pos_v)

    ixs = (ix0, ix1)
    ibs = (ib0, ib1)
    obs = (ob0, ob1)
    si = (si0, si1)
    sg = (sg0, sg1)
    ss = (ss0, ss1)

    def issue_idx(g, b):
        for j in range(G):
            pltpu.async_copy(src_hbm.at[row0 + g * G + j], ixs[b].at[j], si[b])

    def drain_idx(g, b):
        for j in range(G):
            pltpu.make_async_copy(
                src_hbm.at[row0 + g * G + j], ixs[b].at[j], si[b]).wait()

    def issue_gather(g, b):
        del g
        for j in range(G):
            pltpu.async_copy(
                tok_hbm.at[ixs[b].at[j]], ibs[b].at[j], sg[b])

    def drain_gather(g, b):
        del g
        for j in range(G):
            pltpu.make_async_copy(
                tok_hbm.at[ixs[b].at[j]], ibs[b].at[j], sg[b]).wait()

    def issue_scatter(g, b):
        pltpu.async_copy(obs[b], out_hbm.at[pl.ds(row0 + g * G, G)], ss[b])

    def drain_scatter(g, b):
        pltpu.make_async_copy(
            obs[b], out_hbm.at[pl.ds(row0 + g * G, G)], ss[b]).wait()

    def compute(b):
        ib, ob = ibs[b], obs[b]

        def row(r, carry):
            for e in range(NREG):
                pv = pos_v[r, pl.ds(e * LANES, LANES)]
                for j in range(G):
                    tv = ib[j, r, pl.ds(e * LANES, LANES)]
                    ob[j, r, pl.ds(e * LANES, LANES)] = tv * SCALE + pv
            return carry

        lax.fori_loop(0, SEQ, row, 0)

    issue_idx(0, 0)
    issue_idx(1, 1)
    drain_idx(0, 0)
    issue_gather(0, 0)

    def outer(i, carry):
        for b in range(2):
            g = i * 2 + b
            drain_gather(g, b)

            @pl.when(g + 2 < NG)
            def _():
                issue_idx(g + 2, b)

            bo = 1 - b

            @pl.when(g + 1 < NG)
            def _():
                drain_idx(g + 1, bo)
                issue_gather(g + 1, bo)

            @pl.when(g >= 2)
            def _():
                drain_scatter(g - 2, b)

            compute(b)
            issue_scatter(g, b)
        return carry

    lax.fori_loop(0, NG // 2, outer, 0)
    drain_scatter(NG - 2, 0)
    drain_scatter(NG - 1, 1)


@jax.jit
def _emb_lookup(src, token_table, pos_table):
    fn = pl.kernel(
        _body,
        out_type=jax.ShapeDtypeStruct((BATCH, SEQ, EMB), jnp.float32),
        mesh=plsc.VectorSubcoreMesh(core_axis_name="c", subcore_axis_name="s"),
        compiler_params=pltpu.CompilerParams(use_tc_tiling_on_sc=False),
        scratch_types=[
            pltpu.VMEM((G, SEQ), jnp.int32),
            pltpu.VMEM((G, SEQ), jnp.int32),
            pltpu.VMEM((G, SEQ, EMB), jnp.float32),
            pltpu.VMEM((G, SEQ, EMB), jnp.float32),
            pltpu.VMEM((G, SEQ, EMB), jnp.float32),
            pltpu.VMEM((G, SEQ, EMB), jnp.float32),
            pltpu.VMEM((SEQ, EMB), jnp.float32),
            pltpu.SemaphoreType.DMA,
            pltpu.SemaphoreType.DMA,
            pltpu.SemaphoreType.DMA,
            pltpu.SemaphoreType.DMA,
            pltpu.SemaphoreType.DMA,
            pltpu.SemaphoreType.DMA,
        ],
    )
    return fn(src, token_table, pos_table)


def kernel(src, tgt, token_table, pos_table):
    del tgt
    return _emb_lookup(src, token_table, pos_table)
